# baseline (speedup 1.0000x reference)
import functools

import jax
import jax.numpy as jnp
from jax import lax
from jax.experimental import pallas as pl
from jax.experimental.pallas import tpu as pltpu
from jax.experimental.pallas import tpu_sc as plsc

B = 16384
NUM = 13
F = 26
V = 100000
D = 32

NC = 2
NS = 16
NW = NC * NS

BH = B // 2
BQ = B // 4
NGQ = BQ // 16


def _sc_gather_t(table_t, cat_t):
    mesh = plsc.VectorSubcoreMesh(core_axis_name="c", subcore_axis_name="s")

    @functools.partial(
        pl.kernel,
        out_type=jax.ShapeDtypeStruct((F * D, B), jnp.float32),
        mesh=mesh,
        scratch_types=[
            pltpu.VMEM((V,), jnp.float32),
            pltpu.VMEM((BH,), jnp.int32),
            pltpu.VMEM((BH,), jnp.int32),
            pltpu.VMEM((BQ,), jnp.float32),
            pltpu.VMEM((BQ,), jnp.float32),
            pltpu.SemaphoreType.DMA,
            pltpu.SemaphoreType.DMA,
            pltpu.SemaphoreType.DMA,
        ],
        compiler_params=pltpu.CompilerParams(
            use_tc_tiling_on_sc=True, needs_layout_passes=False),
    )
    def gather_kernel(tab_hbm, cat_hbm, out_hbm, row_v, vidx0_v, vidx1_v,
                      stag0_v, stag1_v, sem_row, sem_idx, sem_out):
        wid = lax.axis_index("s") * NC + lax.axis_index("c")
        vidx = (vidx0_v, vidx1_v)
        stag = (stag0_v, stag1_v)

        def field_body(f, _):
            r = f * D + wid
            row_cp = pltpu.async_copy(tab_hbm.at[r], row_v, sem_row)
            idx_cp0 = pltpu.async_copy(
                cat_hbm.at[f, pl.ds(0, BH)], vidx0_v, sem_idx)
            idx_cp1 = pltpu.async_copy(
                cat_hbm.at[f, pl.ds(BH, BH)], vidx1_v, sem_idx)
            idx_cp0.wait()
            row_cp.wait()
            out_cps = []
            for q in range(4):
                h = q // 2
                if q == 2:
                    idx_cp1.wait()
                qoff = (q % 2) * BQ
                vidx_v = vidx[h]
                stag_v = stag[q % 2]

                def grp_body(g, _, vidx_v=vidx_v, stag_v=stag_v, qoff=qoff):
                    sl = pl.ds(qoff + g * 16, 16)
                    dl = pl.ds(g * 16, 16)
                    stag_v[dl] = plsc.load_gather(row_v, [vidx_v[sl]])
                    return _

                if len(out_cps) >= 2:
                    out_cps.pop(0).wait()
                lax.fori_loop(0, NGQ, grp_body, None)
                out_cps.append(pltpu.async_copy(
                    stag_v, out_hbm.at[r, pl.ds(q * BQ, BQ)], sem_out))
            for cp in out_cps:
                cp.wait()
            return _

        lax.fori_loop(0, F, field_body, None)

    return gather_kernel(table_t, cat_t)


_BM = 1024
_INV_SQRT = float(1.0 / (1.0 + 1e-5) ** 0.5)


def _mlp_kernel(num_ref, embt_ref, w1n_ref, w1e_ref, b1_ref, g1_ref, be1_ref,
                w2_ref, b2_ref, g2_ref, be2_ref, wp_ref, bp_ref, out_ref):
    x = jnp.dot(num_ref[...], w1n_ref[...], preferred_element_type=jnp.float32)
    x = x + lax.dot_general(embt_ref[...], w1e_ref[...],
                            (((0,), (0,)), ((), ())),
                            preferred_element_type=jnp.float32)
    x = (x + b1_ref[...]) * (g1_ref[...] * _INV_SQRT) + be1_ref[...]
    x = jnp.maximum(x, 0.0)
    x = jnp.dot(x, w2_ref[...], preferred_element_type=jnp.float32)
    x = (x + b2_ref[...]) * (g2_ref[...] * _INV_SQRT) + be2_ref[...]
    x = jnp.maximum(x, 0.0)
    x = jnp.dot(x, wp_ref[...], preferred_element_type=jnp.float32)
    out_ref[...] = x + bp_ref[...]


def _mlp(numerical, emb_t, W1, b1, g1, be1, W2, b2, g2, be2, Wp, bp):
    W1n = W1[:NUM]
    W1e = W1[NUM:]
    row = lambda v: v.reshape(1, -1)
    grid = (B // _BM,)
    full = lambda shape: pl.BlockSpec(shape, lambda i: (0, 0))
    return pl.pallas_call(
        _mlp_kernel,
        grid=grid,
        in_specs=[
            pl.BlockSpec((_BM, NUM), lambda i: (i, 0)),
            pl.BlockSpec((F * D, _BM), lambda i: (0, i)),
            full((NUM, 512)),
            full((F * D, 512)),
            full((1, 512)), full((1, 512)), full((1, 512)),
            full((512, 256)),
            full((1, 256)), full((1, 256)), full((1, 256)),
            full((256, 768)),
            full((1, 768)),
        ],
        out_specs=pl.BlockSpec((_BM, 768), lambda i: (i, 0)),
        out_shape=jax.ShapeDtypeStruct((B, 768), jnp.float32),
    )(numerical, emb_t, W1n, W1e, row(b1), row(g1), row(be1),
      W2, row(b2), row(g2), row(be2), Wp, row(bp))


def kernel(numerical_data, categorical_data, emb_tables, W1, b1, g1, be1,
           W2, b2, g2, be2, Wp, bp):
    table_t = emb_tables.transpose(0, 2, 1).reshape(F * D, V)
    cat_t = categorical_data.astype(jnp.int32).T
    emb_t = _sc_gather_t(table_t, cat_t)
    return _mlp(numerical_data, emb_t, W1, b1, g1, be1,
                W2, b2, g2, be2, Wp, bp)


# speedup vs baseline: 1.1651x; 1.1651x over previous
import functools

import jax
import jax.numpy as jnp
from jax import lax
from jax.experimental import pallas as pl
from jax.experimental.pallas import tpu as pltpu
from jax.experimental.pallas import tpu_sc as plsc

B = 16384
NUM = 13
F = 26
V = 100000
D = 32

NC = 2
NS = 16
NW = NC * NS

BH = B // 2
NG = BH // 16


def _sc_gather_t(table_t, cat_t):
    mesh = plsc.VectorSubcoreMesh(core_axis_name="c", subcore_axis_name="s")

    @functools.partial(
        pl.kernel,
        out_type=jax.ShapeDtypeStruct((F * D, B), jnp.float32),
        mesh=mesh,
        scratch_types=[
            pltpu.VMEM((V,), jnp.float32),
            pltpu.VMEM((BH,), jnp.int32),
            pltpu.VMEM((BH,), jnp.float32),
            pltpu.SemaphoreType.DMA,
            pltpu.SemaphoreType.DMA,
        ],
        compiler_params=pltpu.CompilerParams(
            use_tc_tiling_on_sc=True, needs_layout_passes=False),
    )
    def gather_kernel(tab_hbm, cat_hbm, out_hbm, row_v, vidx_v, stag_v,
                      sem_row, sem):
        wid = lax.axis_index("s") * NC + lax.axis_index("c")

        def field_body(f, _):
            r = f * D + wid
            row_cp = pltpu.async_copy(tab_hbm.at[r], row_v, sem_row)
            for h in range(2):
                pltpu.sync_copy(cat_hbm.at[f, pl.ds(h * BH, BH)], vidx_v)
                if h == 0:
                    row_cp.wait()

                def grp_body(g, _):
                    for u in range(2):
                        sl = pl.ds(g * 32 + u * 16, 16)
                        stag_v[sl] = plsc.load_gather(row_v, [vidx_v[sl]])
                    return _

                lax.fori_loop(0, NG // 2, grp_body, None)
                pltpu.sync_copy(stag_v, out_hbm.at[r, pl.ds(h * BH, BH)])
            return _

        lax.fori_loop(0, F, field_body, None)

    return gather_kernel(table_t, cat_t)


_BM = 1024
_INV_SQRT = float(1.0 / (1.0 + 1e-5) ** 0.5)


def _mlp_kernel(num_ref, embt_ref, w1n_ref, w1e_ref, b1_ref, g1_ref, be1_ref,
                w2_ref, b2_ref, g2_ref, be2_ref, wp_ref, bp_ref, out_ref):
    x = jnp.dot(num_ref[...], w1n_ref[...], preferred_element_type=jnp.float32)
    x = x + lax.dot_general(embt_ref[...], w1e_ref[...],
                            (((0,), (0,)), ((), ())),
                            preferred_element_type=jnp.float32)
    x = (x + b1_ref[...]) * (g1_ref[...] * _INV_SQRT) + be1_ref[...]
    x = jnp.maximum(x, 0.0)
    x = jnp.dot(x, w2_ref[...], preferred_element_type=jnp.float32)
    x = (x + b2_ref[...]) * (g2_ref[...] * _INV_SQRT) + be2_ref[...]
    x = jnp.maximum(x, 0.0)
    x = jnp.dot(x, wp_ref[...], preferred_element_type=jnp.float32)
    out_ref[...] = x + bp_ref[...]


def _mlp(numerical, emb_t, W1, b1, g1, be1, W2, b2, g2, be2, Wp, bp):
    W1n = W1[:NUM]
    W1e = W1[NUM:]
    row = lambda v: v.reshape(1, -1)
    grid = (B // _BM,)
    full = lambda shape: pl.BlockSpec(shape, lambda i: (0, 0))
    return pl.pallas_call(
        _mlp_kernel,
        grid=grid,
        in_specs=[
            pl.BlockSpec((_BM, NUM), lambda i: (i, 0)),
            pl.BlockSpec((F * D, _BM), lambda i: (0, i)),
            full((NUM, 512)),
            full((F * D, 512)),
            full((1, 512)), full((1, 512)), full((1, 512)),
            full((512, 256)),
            full((1, 256)), full((1, 256)), full((1, 256)),
            full((256, 768)),
            full((1, 768)),
        ],
        out_specs=pl.BlockSpec((_BM, 768), lambda i: (i, 0)),
        out_shape=jax.ShapeDtypeStruct((B, 768), jnp.float32),
    )(numerical, emb_t, W1n, W1e, row(b1), row(g1), row(be1),
      W2, row(b2), row(g2), row(be2), Wp, row(bp))


def kernel(numerical_data, categorical_data, emb_tables, W1, b1, g1, be1,
           W2, b2, g2, be2, Wp, bp):
    table_t = emb_tables.transpose(0, 2, 1).reshape(F * D, V)
    cat_t = categorical_data.astype(jnp.int32).T
    emb_t = _sc_gather_t(table_t, cat_t)
    return _mlp(numerical_data, emb_t, W1, b1, g1, be1,
                W2, b2, g2, be2, Wp, bp)
